# baseline (speedup 1.0000x reference)
import jax
import jax.numpy as jnp
from jax.experimental import pallas as pl
from jax.experimental.pallas import tpu as pltpu

_VPAD = 128
_EMB = 128
_HID2 = 512
_OUT = 256

_NT = (((1,), (1,)), ((), ()))


def _fused_kernel(idx_ref, meter_ref, unit_ref, w1_ref, b1_ref, w2_ref, b2_ref,
                  out_ref, ab_ref, w2t_ref, b1c_ref):
    @pl.when(pl.program_id(0) == 0)
    def _prep():
        a = jax.lax.dot_general(meter_ref[...], w1_ref[:, :_EMB], _NT,
                                preferred_element_type=jnp.float32)
        b = jax.lax.dot_general(unit_ref[...], w1_ref[:, _EMB:], _NT,
                                preferred_element_type=jnp.float32)
        npad = _VPAD - a.shape[0]
        ab_ref[...] = jnp.concatenate(
            [jnp.pad(a, ((0, npad), (0, 0))), jnp.pad(b, ((0, npad), (0, 0)))],
            axis=0).astype(jnp.bfloat16)
        w2t_ref[...] = w2_ref[...].T.astype(jnp.bfloat16)
        b1c_ref[...] = b1_ref[...].astype(jnp.bfloat16)

    bn = idx_ref.shape[0]
    idx2 = idx_ref[:, 2:3]
    idx3 = idx_ref[:, 3:4]
    iota2 = jax.lax.broadcasted_iota(jnp.int32, (bn, 2, _VPAD), 2)
    iota128 = iota2.reshape(bn, 2 * _VPAD)
    half = jax.lax.broadcasted_iota(jnp.int32, (bn, 2 * _VPAD), 1) >= _VPAD
    sel = jnp.where(half, idx3, idx2)
    oh = (iota128 == sel).astype(jnp.bfloat16)
    h = jnp.dot(oh, ab_ref[...], preferred_element_type=jnp.float32)
    h = jnp.maximum(h.astype(jnp.bfloat16) + b1c_ref[...], jnp.bfloat16(0.0))
    out_ref[...] = (
        jnp.dot(h, w2t_ref[...], preferred_element_type=jnp.float32) + b2_ref[...]
    )


def kernel(input_tensor, meter_table, unit_table, W1, b1, W2, b2):
    n = input_tensor.shape[0]
    bn = 2048
    v_meter = meter_table.shape[0]
    v_unit = unit_table.shape[0]

    out = pl.pallas_call(
        _fused_kernel,
        grid=(n // bn,),
        in_specs=[
            pl.BlockSpec((bn, 4), lambda i: (i, 0)),
            pl.BlockSpec((v_meter, _EMB), lambda i: (0, 0)),
            pl.BlockSpec((v_unit, _EMB), lambda i: (0, 0)),
            pl.BlockSpec((_HID2, 2 * _EMB), lambda i: (0, 0)),
            pl.BlockSpec((1, _HID2), lambda i: (0, 0)),
            pl.BlockSpec((_OUT, _HID2), lambda i: (0, 0)),
            pl.BlockSpec((1, _OUT), lambda i: (0, 0)),
        ],
        out_specs=pl.BlockSpec((bn, _OUT), lambda i: (i, 0)),
        out_shape=jax.ShapeDtypeStruct((n, _OUT), jnp.float32),
        scratch_shapes=[
            pltpu.VMEM((2 * _VPAD, _HID2), jnp.bfloat16),
            pltpu.VMEM((_HID2, _OUT), jnp.bfloat16),
            pltpu.VMEM((1, _HID2), jnp.bfloat16),
        ],
    )(input_tensor, meter_table, unit_table, W1,
      b1.reshape(1, _HID2), W2, b2.reshape(1, _OUT))
    return out


# speedup vs baseline: 1.1406x; 1.1406x over previous
import jax
import jax.numpy as jnp
from jax.experimental import pallas as pl
from jax.experimental.pallas import tpu as pltpu

_VPAD = 128
_EMB = 128
_HID2 = 512
_OUT = 256

_NT = (((1,), (1,)), ((), ()))


def _fused_kernel(idx_ref, meter_ref, unit_ref, w1_ref, b1_ref, w2_ref, b2_ref,
                  out_ref, ab_ref, w2t_ref, b1c_ref, iota_ref):
    @pl.when(pl.program_id(0) == 0)
    def _prep():
        a = jax.lax.dot_general(meter_ref[...], w1_ref[:, :_EMB], _NT,
                                preferred_element_type=jnp.float32)
        b = jax.lax.dot_general(unit_ref[...], w1_ref[:, _EMB:], _NT,
                                preferred_element_type=jnp.float32)
        npad = _VPAD - a.shape[0]
        ab_ref[...] = jnp.concatenate(
            [jnp.pad(a, ((0, npad), (0, 0))), jnp.pad(b, ((0, npad), (0, 0)))],
            axis=0).astype(jnp.bfloat16)
        w2t_ref[...] = w2_ref[...].T.astype(jnp.bfloat16)
        b1c_ref[...] = b1_ref[...].astype(jnp.bfloat16)
        iota_ref[...] = jax.lax.broadcasted_iota(
            jnp.int32, (8, _VPAD), 1).astype(jnp.bfloat16)

    bn = idx_ref.shape[0]
    idx2b = idx_ref[:, 2:3].astype(jnp.bfloat16)
    idx3b = idx_ref[:, 3:4].astype(jnp.bfloat16)
    iota_b = jnp.broadcast_to(iota_ref[0:1, :], (bn, _VPAD))
    one = jnp.bfloat16(1.0)
    zero = jnp.bfloat16(0.0)
    oh = jnp.concatenate(
        [jnp.where(iota_b == idx2b, one, zero),
         jnp.where(iota_b == idx3b, one, zero)], axis=1)
    h = jnp.dot(oh, ab_ref[...], preferred_element_type=jnp.float32)
    h = jnp.maximum(h.astype(jnp.bfloat16) + b1c_ref[...], jnp.bfloat16(0.0))
    out_ref[...] = (
        jnp.dot(h, w2t_ref[...], preferred_element_type=jnp.float32) + b2_ref[...]
    )


def kernel(input_tensor, meter_table, unit_table, W1, b1, W2, b2):
    n = input_tensor.shape[0]
    bn = 4096
    v_meter = meter_table.shape[0]
    v_unit = unit_table.shape[0]

    out = pl.pallas_call(
        _fused_kernel,
        grid=(n // bn,),
        in_specs=[
            pl.BlockSpec((bn, 4), lambda i: (i, 0)),
            pl.BlockSpec((v_meter, _EMB), lambda i: (0, 0)),
            pl.BlockSpec((v_unit, _EMB), lambda i: (0, 0)),
            pl.BlockSpec((_HID2, 2 * _EMB), lambda i: (0, 0)),
            pl.BlockSpec((1, _HID2), lambda i: (0, 0)),
            pl.BlockSpec((_OUT, _HID2), lambda i: (0, 0)),
            pl.BlockSpec((1, _OUT), lambda i: (0, 0)),
        ],
        out_specs=pl.BlockSpec((bn, _OUT), lambda i: (i, 0)),
        out_shape=jax.ShapeDtypeStruct((n, _OUT), jnp.float32),
        scratch_shapes=[
            pltpu.VMEM((2 * _VPAD, _HID2), jnp.bfloat16),
            pltpu.VMEM((_HID2, _OUT), jnp.bfloat16),
            pltpu.VMEM((1, _HID2), jnp.bfloat16),
            pltpu.VMEM((8, _VPAD), jnp.bfloat16),
        ],
    )(input_tensor, meter_table, unit_table, W1,
      b1.reshape(1, _HID2), W2, b2.reshape(1, _OUT))
    return out
